# baseline (device time: 209063 ns/iter reference)
import jax
import jax.numpy as jnp
from jax import lax
from jax.experimental import pallas as pl
from jax.experimental.pallas import tpu as pltpu

B, SQ, H, D = 8, 8, 16, 128
SCALE = D ** -0.5


def _flash_partial(Q3, K3, V3, skv):
    KC = 512
    nkc = skv // KC

    def body(q_ref, k_ref, v_ref, num_ref, l_ref, p_ref):
        kc = pl.program_id(1)

        @pl.when(kc == 0)
        def _():
            num_ref[...] = jnp.zeros_like(num_ref)
            l_ref[...] = jnp.zeros_like(l_ref)

        for h in range(H):
            sl = slice(h * D, (h + 1) * D)
            psl = slice(h * KC, (h + 1) * KC)
            q = q_ref[:, sl].astype(jnp.bfloat16)
            k = k_ref[:, sl].astype(jnp.bfloat16)
            p_ref[:, psl] = lax.dot_general(
                q, k, (((1,), (1,)), ((), ())),
                preferred_element_type=jnp.float32,
            )
        p_ref[...] = jnp.exp(p_ref[...] * SCALE)
        for h in range(H):
            sl = slice(h * D, (h + 1) * D)
            psl = slice(h * KC, (h + 1) * KC)
            p = p_ref[:, psl]
            l_ref[h] += jnp.sum(p, axis=1, keepdims=True)
            num_ref[:, sl] += lax.dot_general(
                p.astype(jnp.bfloat16), v_ref[:, sl].astype(jnp.bfloat16),
                (((1,), (0,)), ((), ())),
                preferred_element_type=jnp.float32,
            )

    return pl.pallas_call(
        body,
        grid=(B, nkc),
        in_specs=[
            pl.BlockSpec((None, SQ, H * D), lambda b, kc: (b, 0, 0)),
            pl.BlockSpec((None, KC, H * D), lambda b, kc: (b, kc, 0)),
            pl.BlockSpec((None, KC, H * D), lambda b, kc: (b, kc, 0)),
        ],
        out_specs=[
            pl.BlockSpec((None, SQ, H * D), lambda b, kc: (b, 0, 0)),
            pl.BlockSpec((None, H, SQ, 1), lambda b, kc: (b, 0, 0, 0)),
        ],
        out_shape=[
            jax.ShapeDtypeStruct((B, SQ, H * D), jnp.float32),
            jax.ShapeDtypeStruct((B, H, SQ, 1), jnp.float32),
        ],
        scratch_shapes=[
            pltpu.VMEM((SQ, H * KC), jnp.float32),
        ],
    )(Q3, K3, V3)


def _exchange_sum(num, l):

    def body(num_ref, l_ref, nsum_ref, lsum_ref, rnum_ref, rl_ref,
             nsend, nrecv, lsend, lrecv):
        my_x = lax.axis_index("x")
        my_y = lax.axis_index("y")
        my_z = lax.axis_index("z")
        peer = (1 - my_x, my_y, my_z)

        barrier = pltpu.get_barrier_semaphore()
        pl.semaphore_signal(
            barrier, inc=1, device_id=peer,
            device_id_type=pl.DeviceIdType.MESH,
        )
        pl.semaphore_wait(barrier, 1)

        rdma_n = pltpu.make_async_remote_copy(
            src_ref=num_ref, dst_ref=rnum_ref,
            send_sem=nsend, recv_sem=nrecv,
            device_id=peer, device_id_type=pl.DeviceIdType.MESH,
        )
        rdma_l = pltpu.make_async_remote_copy(
            src_ref=l_ref, dst_ref=rl_ref,
            send_sem=lsend, recv_sem=lrecv,
            device_id=peer, device_id_type=pl.DeviceIdType.MESH,
        )
        rdma_n.start()
        rdma_l.start()
        rdma_n.wait()
        rdma_l.wait()

        nsum_ref[...] = num_ref[...] + rnum_ref[...]
        lsum_ref[...] = l_ref[...] + rl_ref[...]

    return pl.pallas_call(
        body,
        in_specs=[
            pl.BlockSpec(memory_space=pltpu.VMEM),
            pl.BlockSpec(memory_space=pltpu.VMEM),
        ],
        out_specs=[
            pl.BlockSpec(memory_space=pltpu.VMEM),
            pl.BlockSpec(memory_space=pltpu.VMEM),
        ],
        out_shape=[
            jax.ShapeDtypeStruct((B, SQ, H * D), jnp.float32),
            jax.ShapeDtypeStruct((B, H, SQ, 1), jnp.float32),
        ],
        scratch_shapes=[
            pltpu.VMEM((B, SQ, H * D), jnp.float32),
            pltpu.VMEM((B, H, SQ, 1), jnp.float32),
            pltpu.SemaphoreType.DMA,
            pltpu.SemaphoreType.DMA,
            pltpu.SemaphoreType.DMA,
            pltpu.SemaphoreType.DMA,
        ],
        compiler_params=pltpu.CompilerParams(collective_id=0),
    )(num, l)


def _divide(nsum, lsum):

    def body(n_ref, l_ref, out_ref):
        out_ref[...] = n_ref[...] / l_ref[...]

    return pl.pallas_call(
        body,
        grid=(B, H),
        in_specs=[
            pl.BlockSpec((None, SQ, D), lambda b, h: (b, 0, h)),
            pl.BlockSpec((None, None, SQ, 1), lambda b, h: (b, h, 0, 0)),
        ],
        out_specs=pl.BlockSpec((None, SQ, D), lambda b, h: (b, 0, h)),
        out_shape=jax.ShapeDtypeStruct((B, SQ, H * D), jnp.float32),
    )(nsum, lsum)


def kernel(Q, K, V):
    skv = K.shape[1]
    Q3 = Q.reshape(B, SQ, H * D)
    K3 = K.reshape(B, skv, H * D)
    V3 = V.reshape(B, skv, H * D)
    num, l = _flash_partial(Q3, K3, V3, skv)
    nsum, lsum = _exchange_sum(num, l)
    out = _divide(nsum, lsum)
    return out.reshape(B, SQ, H, D)


# device time: 67774 ns/iter; 3.0847x vs baseline; 3.0847x over previous
import jax
import jax.numpy as jnp
from jax import lax
from jax.experimental import pallas as pl
from jax.experimental.pallas import tpu as pltpu

B, SQ, H, D = 8, 8, 16, 128
QH = SQ * H
SCALE = D ** -0.5


def _flash_partial(Q, K, V):
    skv = K.shape[1]
    KC = 512
    nkc = skv // KC
    RH = KC * H

    def body(q_ref, k_ref, v_ref, num_ref, l_ref, pen_ref, p_ref, ones_ref):
        b = pl.program_id(0)
        kc = pl.program_id(1)

        @pl.when((b == 0) & (kc == 0))
        def _():
            r = lax.broadcasted_iota(jnp.int32, (RH, QH), 0)
            c = lax.broadcasted_iota(jnp.int32, (RH, QH), 1)
            pen_ref[...] = jnp.where((r % H) == (c % H), 0.0, -1e9).astype(
                jnp.float32
            )
            ones_ref[...] = jnp.ones((RH, 8), jnp.bfloat16)

        @pl.when(kc == 0)
        def _():
            num_ref[...] = jnp.zeros_like(num_ref)
            l_ref[...] = jnp.zeros_like(l_ref)

        q = q_ref[...].reshape(QH, D).astype(jnp.bfloat16)
        k = k_ref[...].reshape(RH, D).astype(jnp.bfloat16)
        g = lax.dot_general(
            k, q, (((1,), (1,)), ((), ())),
            preferred_element_type=jnp.float32,
        )
        p_ref[...] = jnp.exp(g * SCALE + pen_ref[...]).astype(jnp.bfloat16)

        p = p_ref[...]
        v = v_ref[...].reshape(RH, D).astype(jnp.bfloat16)
        num_ref[...] += lax.dot_general(
            p, v, (((0,), (0,)), ((), ())),
            preferred_element_type=jnp.float32,
        ).reshape(SQ, H, D)
        l_ref[...] += lax.dot_general(
            p, ones_ref[...], (((0,), (0,)), ((), ())),
            preferred_element_type=jnp.float32,
        )[:, 0:1]

    return pl.pallas_call(
        body,
        grid=(B, nkc),
        in_specs=[
            pl.BlockSpec((None, SQ, H, D), lambda b, kc: (b, 0, 0, 0)),
            pl.BlockSpec((None, KC, H, D), lambda b, kc: (b, kc, 0, 0)),
            pl.BlockSpec((None, KC, H, D), lambda b, kc: (b, kc, 0, 0)),
        ],
        out_specs=[
            pl.BlockSpec((None, SQ, H, D), lambda b, kc: (b, 0, 0, 0)),
            pl.BlockSpec((None, QH, 1), lambda b, kc: (b, 0, 0)),
        ],
        out_shape=[
            jax.ShapeDtypeStruct((B, SQ, H, D), jnp.float32),
            jax.ShapeDtypeStruct((B, QH, 1), jnp.float32),
        ],
        scratch_shapes=[
            pltpu.VMEM((RH, QH), jnp.float32),
            pltpu.VMEM((RH, QH), jnp.bfloat16),
            pltpu.VMEM((RH, 8), jnp.bfloat16),
        ],
    )(Q, K, V)


def _exchange_combine(num, l):

    def body(num_ref, l_ref, out_ref, rnum_ref, rl_ref,
             nsend, nrecv, lsend, lrecv):
        my_x = lax.axis_index("x")
        my_y = lax.axis_index("y")
        my_z = lax.axis_index("z")
        peer = (1 - my_x, my_y, my_z)

        barrier = pltpu.get_barrier_semaphore()
        pl.semaphore_signal(
            barrier, inc=1, device_id=peer,
            device_id_type=pl.DeviceIdType.MESH,
        )
        pl.semaphore_wait(barrier, 1)

        rdma_n = pltpu.make_async_remote_copy(
            src_ref=num_ref, dst_ref=rnum_ref,
            send_sem=nsend, recv_sem=nrecv,
            device_id=peer, device_id_type=pl.DeviceIdType.MESH,
        )
        rdma_l = pltpu.make_async_remote_copy(
            src_ref=l_ref, dst_ref=rl_ref,
            send_sem=lsend, recv_sem=lrecv,
            device_id=peer, device_id_type=pl.DeviceIdType.MESH,
        )
        rdma_n.start()
        rdma_l.start()
        rdma_n.wait()
        rdma_l.wait()

        nsum = (num_ref[...] + rnum_ref[...]).reshape(B, QH, D)
        lsum = l_ref[...] + rl_ref[...]
        out_ref[...] = (nsum / lsum).reshape(B, SQ, H, D)

    return pl.pallas_call(
        body,
        in_specs=[
            pl.BlockSpec(memory_space=pltpu.VMEM),
            pl.BlockSpec(memory_space=pltpu.VMEM),
        ],
        out_specs=pl.BlockSpec(memory_space=pltpu.VMEM),
        out_shape=jax.ShapeDtypeStruct((B, SQ, H, D), jnp.float32),
        scratch_shapes=[
            pltpu.VMEM((B, SQ, H, D), jnp.float32),
            pltpu.VMEM((B, QH, 1), jnp.float32),
            pltpu.SemaphoreType.DMA,
            pltpu.SemaphoreType.DMA,
            pltpu.SemaphoreType.DMA,
            pltpu.SemaphoreType.DMA,
        ],
        compiler_params=pltpu.CompilerParams(collective_id=0),
    )(num, l)


def kernel(Q, K, V):
    num, l = _flash_partial(Q, K, V)
    return _exchange_combine(num, l)


# device time: 58294 ns/iter; 3.5864x vs baseline; 1.1626x over previous
import jax
import jax.numpy as jnp
from jax import lax
from jax.experimental import pallas as pl
from jax.experimental.pallas import tpu as pltpu

B, SQ, H, D = 8, 8, 16, 128
QH = SQ * H
SCALE = D ** -0.5


def _flash_partial(Q, K, V):
    skv = K.shape[1]
    KC = 512
    nkc = skv // KC
    RH = KC * H

    def body(q_ref, k_ref, v_ref, num_ref, l_ref, pen_ref, p_ref, ones_ref):
        b = pl.program_id(0)
        kc = pl.program_id(1)

        @pl.when((b == 0) & (kc == 0))
        def _():
            r = lax.broadcasted_iota(jnp.int32, (RH, QH), 0)
            c = lax.broadcasted_iota(jnp.int32, (RH, QH), 1)
            pen_ref[...] = jnp.where((r % H) == (c % H), 0.0, -1e9).astype(
                jnp.float32
            )
            ones_ref[...] = jnp.ones((RH, 8), jnp.bfloat16)

        @pl.when(kc == 0)
        def _():
            num_ref[...] = jnp.zeros_like(num_ref)
            l_ref[...] = jnp.zeros_like(l_ref)

        q = q_ref[...].reshape(QH, D).astype(jnp.bfloat16)
        k = k_ref[...].reshape(RH, D).astype(jnp.bfloat16)
        g = lax.dot_general(
            k, q, (((1,), (1,)), ((), ())),
            preferred_element_type=jnp.float32,
        )
        p_ref[...] = jnp.exp(g * SCALE + pen_ref[...]).astype(jnp.bfloat16)

        p = p_ref[...]
        v = v_ref[...].reshape(RH, D).astype(jnp.bfloat16)
        num_ref[...] += lax.dot_general(
            p, v, (((0,), (0,)), ((), ())),
            preferred_element_type=jnp.float32,
        ).reshape(SQ, H, D)
        l_ref[...] += lax.dot_general(
            p, ones_ref[...], (((0,), (0,)), ((), ())),
            preferred_element_type=jnp.float32,
        )[:, 0:1]

    return pl.pallas_call(
        body,
        grid=(B, nkc),
        in_specs=[
            pl.BlockSpec((None, SQ, H, D), lambda b, kc: (b, 0, 0, 0)),
            pl.BlockSpec((None, KC, H, D), lambda b, kc: (b, kc, 0, 0)),
            pl.BlockSpec((None, KC, H, D), lambda b, kc: (b, kc, 0, 0)),
        ],
        out_specs=[
            pl.BlockSpec((None, SQ, H, D), lambda b, kc: (b, 0, 0, 0)),
            pl.BlockSpec((None, QH, 1), lambda b, kc: (b, 0, 0)),
        ],
        out_shape=[
            jax.ShapeDtypeStruct((B, SQ, H, D), jnp.float32),
            jax.ShapeDtypeStruct((B, QH, 1), jnp.float32),
        ],
        scratch_shapes=[
            pltpu.VMEM((RH, QH), jnp.float32),
            pltpu.VMEM((RH, QH), jnp.bfloat16),
            pltpu.VMEM((RH, 8), jnp.bfloat16),
        ],
    )(Q, K, V)


def _exchange_combine(num, l):

    def body(num_ref, l_ref, out_ref, rnum_ref, rl_ref,
             nsend, nrecv, lsend, lrecv):
        my_x = lax.axis_index("x")
        my_y = lax.axis_index("y")
        my_z = lax.axis_index("z")
        peer = (1 - my_x, my_y, my_z)

        barrier = pltpu.get_barrier_semaphore()
        pl.semaphore_signal(
            barrier, inc=1, device_id=peer,
            device_id_type=pl.DeviceIdType.MESH,
        )
        pl.semaphore_wait(barrier, 1)

        rdma_n = pltpu.make_async_remote_copy(
            src_ref=num_ref, dst_ref=rnum_ref,
            send_sem=nsend, recv_sem=nrecv,
            device_id=peer, device_id_type=pl.DeviceIdType.MESH,
        )
        rdma_l = pltpu.make_async_remote_copy(
            src_ref=l_ref, dst_ref=rl_ref,
            send_sem=lsend, recv_sem=lrecv,
            device_id=peer, device_id_type=pl.DeviceIdType.MESH,
        )
        rdma_n.start()
        rdma_l.start()
        rdma_n.wait()
        rdma_l.wait()

        nsum = (num_ref[...] + rnum_ref[...]).reshape(B, QH, D)
        lsum = l_ref[...] + rl_ref[...]
        out_ref[...] = (nsum / lsum).reshape(B, SQ, H, D)

    return pl.pallas_call(
        body,
        in_specs=[
            pl.BlockSpec(memory_space=pltpu.VMEM),
            pl.BlockSpec(memory_space=pltpu.VMEM),
        ],
        out_specs=pl.BlockSpec(memory_space=pltpu.VMEM),
        out_shape=jax.ShapeDtypeStruct((B, SQ, H, D), jnp.float32),
        scratch_shapes=[
            pltpu.VMEM((B, SQ, H, D), jnp.float32),
            pltpu.VMEM((B, QH, 1), jnp.float32),
            pltpu.SemaphoreType.DMA,
            pltpu.SemaphoreType.DMA,
            pltpu.SemaphoreType.DMA,
            pltpu.SemaphoreType.DMA,
        ],
        compiler_params=pltpu.CompilerParams(collective_id=0),
    )(num, l)


def _fused(Q, K, V):
    skv = K.shape[1]
    KC = 512
    nkc = skv // KC
    RH = KC * H

    def body(q_ref, k_ref, v_ref, out_ref,
             pen_ref, p_ref, ones_ref, num_ref, l_ref, rnum_ref, rl_ref,
             nsend, nrecv, lsend, lrecv):
        b = pl.program_id(0)
        kc = pl.program_id(1)
        my_x = lax.axis_index("x")
        my_y = lax.axis_index("y")
        my_z = lax.axis_index("z")
        peer = (1 - my_x, my_y, my_z)

        @pl.when((b == 0) & (kc == 0))
        def _():
            r = lax.broadcasted_iota(jnp.int32, (RH, QH), 0)
            c = lax.broadcasted_iota(jnp.int32, (RH, QH), 1)
            pen_ref[...] = jnp.where((r % H) == (c % H), 0.0, -1e9).astype(
                jnp.float32
            )
            ones_ref[...] = jnp.ones((RH, 8), jnp.bfloat16)
            barrier = pltpu.get_barrier_semaphore()
            pl.semaphore_signal(
                barrier, inc=1, device_id=peer,
                device_id_type=pl.DeviceIdType.MESH,
            )
            pl.semaphore_wait(barrier, 1)

        q = q_ref[...].reshape(QH, D).astype(jnp.bfloat16)
        k = k_ref[...].reshape(RH, D).astype(jnp.bfloat16)
        g = lax.dot_general(
            k, q, (((1,), (1,)), ((), ())),
            preferred_element_type=jnp.float32,
        )
        p_ref[...] = jnp.exp(g * SCALE + pen_ref[...]).astype(jnp.bfloat16)

        p = p_ref[...]
        v = v_ref[...].reshape(RH, D).astype(jnp.bfloat16)
        contrib_n = lax.dot_general(
            p, v, (((0,), (0,)), ((), ())),
            preferred_element_type=jnp.float32,
        ).reshape(SQ, H, D)
        contrib_l = lax.dot_general(
            p, ones_ref[...], (((0,), (0,)), ((), ())),
            preferred_element_type=jnp.float32,
        )[:, 0:1]

        @pl.when(kc == 0)
        def _():
            num_ref[b] = contrib_n
            l_ref[b] = contrib_l

        @pl.when(kc != 0)
        def _():
            num_ref[b] += contrib_n
            l_ref[b] += contrib_l

        def _rdma_pair(bb):
            rn = pltpu.make_async_remote_copy(
                src_ref=num_ref.at[bb], dst_ref=rnum_ref.at[bb],
                send_sem=nsend.at[bb], recv_sem=nrecv.at[bb],
                device_id=peer, device_id_type=pl.DeviceIdType.MESH,
            )
            rl_ = pltpu.make_async_remote_copy(
                src_ref=l_ref.at[bb], dst_ref=rl_ref.at[bb],
                send_sem=lsend.at[bb], recv_sem=lrecv.at[bb],
                device_id=peer, device_id_type=pl.DeviceIdType.MESH,
            )
            return rn, rl_

        @pl.when(kc == nkc - 1)
        def _():
            rn, rl_ = _rdma_pair(b)
            rn.start()
            rl_.start()

        @pl.when((b == B - 1) & (kc == nkc - 1))
        def _():
            for bb in range(B):
                rn, rl_ = _rdma_pair(bb)
                rn.wait_send()
                rl_.wait_send()
                rn.wait_recv()
                rl_.wait_recv()
            nsum = (num_ref[...] + rnum_ref[...]).reshape(B, QH, D)
            lsum = l_ref[...] + rl_ref[...]
            out_ref[...] = (nsum / lsum).reshape(B, SQ, H, D)

    return pl.pallas_call(
        body,
        grid=(B, nkc),
        in_specs=[
            pl.BlockSpec((None, SQ, H, D), lambda b, kc: (b, 0, 0, 0)),
            pl.BlockSpec((None, KC, H, D), lambda b, kc: (b, kc, 0, 0)),
            pl.BlockSpec((None, KC, H, D), lambda b, kc: (b, kc, 0, 0)),
        ],
        out_specs=pl.BlockSpec(memory_space=pltpu.VMEM),
        out_shape=jax.ShapeDtypeStruct((B, SQ, H, D), jnp.float32),
        scratch_shapes=[
            pltpu.VMEM((RH, QH), jnp.float32),
            pltpu.VMEM((RH, QH), jnp.bfloat16),
            pltpu.VMEM((RH, 8), jnp.bfloat16),
            pltpu.VMEM((B, SQ, H, D), jnp.float32),
            pltpu.VMEM((B, QH, 1), jnp.float32),
            pltpu.VMEM((B, SQ, H, D), jnp.float32),
            pltpu.VMEM((B, QH, 1), jnp.float32),
            pltpu.SemaphoreType.DMA((B,)),
            pltpu.SemaphoreType.DMA((B,)),
            pltpu.SemaphoreType.DMA((B,)),
            pltpu.SemaphoreType.DMA((B,)),
        ],
        compiler_params=pltpu.CompilerParams(collective_id=0),
    )(Q, K, V)


def kernel(Q, K, V):
    return _fused(Q, K, V)


# device time: 57776 ns/iter; 3.6185x vs baseline; 1.0090x over previous
import jax
import jax.numpy as jnp
from jax import lax
from jax.experimental import pallas as pl
from jax.experimental.pallas import tpu as pltpu

B, SQ, H, D = 8, 8, 16, 128
QH = SQ * H
SCALE = D ** -0.5


def _flash_partial(Q, K, V):
    skv = K.shape[1]
    KC = 512
    nkc = skv // KC
    RH = KC * H

    def body(q_ref, k_ref, v_ref, num_ref, l_ref, pen_ref, p_ref, ones_ref):
        b = pl.program_id(0)
        kc = pl.program_id(1)

        @pl.when((b == 0) & (kc == 0))
        def _():
            r = lax.broadcasted_iota(jnp.int32, (RH, QH), 0)
            c = lax.broadcasted_iota(jnp.int32, (RH, QH), 1)
            pen_ref[...] = jnp.where((r % H) == (c % H), 0.0, -1e9).astype(
                jnp.float32
            )
            ones_ref[...] = jnp.ones((RH, 8), jnp.bfloat16)

        @pl.when(kc == 0)
        def _():
            num_ref[...] = jnp.zeros_like(num_ref)
            l_ref[...] = jnp.zeros_like(l_ref)

        q = q_ref[...].reshape(QH, D).astype(jnp.bfloat16)
        k = k_ref[...].reshape(RH, D).astype(jnp.bfloat16)
        g = lax.dot_general(
            k, q, (((1,), (1,)), ((), ())),
            preferred_element_type=jnp.float32,
        )
        p_ref[...] = jnp.exp(g * SCALE + pen_ref[...]).astype(jnp.bfloat16)

        p = p_ref[...]
        v = v_ref[...].reshape(RH, D).astype(jnp.bfloat16)
        num_ref[...] += lax.dot_general(
            p, v, (((0,), (0,)), ((), ())),
            preferred_element_type=jnp.float32,
        ).reshape(SQ, H, D)
        l_ref[...] += lax.dot_general(
            p, ones_ref[...], (((0,), (0,)), ((), ())),
            preferred_element_type=jnp.float32,
        )[:, 0:1]

    return pl.pallas_call(
        body,
        grid=(B, nkc),
        in_specs=[
            pl.BlockSpec((None, SQ, H, D), lambda b, kc: (b, 0, 0, 0)),
            pl.BlockSpec((None, KC, H, D), lambda b, kc: (b, kc, 0, 0)),
            pl.BlockSpec((None, KC, H, D), lambda b, kc: (b, kc, 0, 0)),
        ],
        out_specs=[
            pl.BlockSpec((None, SQ, H, D), lambda b, kc: (b, 0, 0, 0)),
            pl.BlockSpec((None, QH, 1), lambda b, kc: (b, 0, 0)),
        ],
        out_shape=[
            jax.ShapeDtypeStruct((B, SQ, H, D), jnp.float32),
            jax.ShapeDtypeStruct((B, QH, 1), jnp.float32),
        ],
        scratch_shapes=[
            pltpu.VMEM((RH, QH), jnp.float32),
            pltpu.VMEM((RH, QH), jnp.bfloat16),
            pltpu.VMEM((RH, 8), jnp.bfloat16),
        ],
    )(Q, K, V)


def _exchange_combine(num, l):

    def body(num_ref, l_ref, out_ref, rnum_ref, rl_ref,
             nsend, nrecv, lsend, lrecv):
        my_x = lax.axis_index("x")
        my_y = lax.axis_index("y")
        my_z = lax.axis_index("z")
        peer = (1 - my_x, my_y, my_z)

        barrier = pltpu.get_barrier_semaphore()
        pl.semaphore_signal(
            barrier, inc=1, device_id=peer,
            device_id_type=pl.DeviceIdType.MESH,
        )
        pl.semaphore_wait(barrier, 1)

        rdma_n = pltpu.make_async_remote_copy(
            src_ref=num_ref, dst_ref=rnum_ref,
            send_sem=nsend, recv_sem=nrecv,
            device_id=peer, device_id_type=pl.DeviceIdType.MESH,
        )
        rdma_l = pltpu.make_async_remote_copy(
            src_ref=l_ref, dst_ref=rl_ref,
            send_sem=lsend, recv_sem=lrecv,
            device_id=peer, device_id_type=pl.DeviceIdType.MESH,
        )
        rdma_n.start()
        rdma_l.start()
        rdma_n.wait()
        rdma_l.wait()

        nsum = (num_ref[...] + rnum_ref[...]).reshape(B, QH, D)
        lsum = l_ref[...] + rl_ref[...]
        out_ref[...] = (nsum / lsum).reshape(B, SQ, H, D)

    return pl.pallas_call(
        body,
        in_specs=[
            pl.BlockSpec(memory_space=pltpu.VMEM),
            pl.BlockSpec(memory_space=pltpu.VMEM),
        ],
        out_specs=pl.BlockSpec(memory_space=pltpu.VMEM),
        out_shape=jax.ShapeDtypeStruct((B, SQ, H, D), jnp.float32),
        scratch_shapes=[
            pltpu.VMEM((B, SQ, H, D), jnp.float32),
            pltpu.VMEM((B, QH, 1), jnp.float32),
            pltpu.SemaphoreType.DMA,
            pltpu.SemaphoreType.DMA,
            pltpu.SemaphoreType.DMA,
            pltpu.SemaphoreType.DMA,
        ],
        compiler_params=pltpu.CompilerParams(collective_id=0),
    )(num, l)


def _fused(Q, K, V):
    skv = K.shape[1]
    KC = 1024
    nkc = skv // KC
    RH = KC * H

    def body(q_ref, k_ref, v_ref, out_ref,
             pen_ref, p_ref, ones_ref, num_ref, l_ref, rnum_ref, rl_ref,
             nsend, nrecv, lsend, lrecv):
        b = pl.program_id(0)
        kc = pl.program_id(1)
        my_x = lax.axis_index("x")
        my_y = lax.axis_index("y")
        my_z = lax.axis_index("z")
        peer = (1 - my_x, my_y, my_z)

        @pl.when((b == 0) & (kc == 0))
        def _():
            r = lax.broadcasted_iota(jnp.int32, (RH, QH), 0)
            c = lax.broadcasted_iota(jnp.int32, (RH, QH), 1)
            pen_ref[...] = jnp.where((r % H) == (c % H), 0.0, -1e9).astype(
                jnp.bfloat16
            )
            ones_ref[...] = jnp.ones((RH, 8), jnp.bfloat16)
            barrier = pltpu.get_barrier_semaphore()
            pl.semaphore_signal(
                barrier, inc=1, device_id=peer,
                device_id_type=pl.DeviceIdType.MESH,
            )
            pl.semaphore_wait(barrier, 1)

        q = (q_ref[...].reshape(QH, D) * SCALE).astype(jnp.bfloat16)
        k = k_ref[...].reshape(RH, D).astype(jnp.bfloat16)
        g = lax.dot_general(
            k, q, (((1,), (1,)), ((), ())),
            preferred_element_type=jnp.float32,
        )
        p_ref[...] = jnp.exp(g.astype(jnp.bfloat16) + pen_ref[...])

        p = p_ref[...]
        v = v_ref[...].reshape(RH, D).astype(jnp.bfloat16)
        contrib_n = lax.dot_general(
            p, v, (((0,), (0,)), ((), ())),
            preferred_element_type=jnp.float32,
        ).reshape(SQ, H, D)
        contrib_l = lax.dot_general(
            p, ones_ref[...], (((0,), (0,)), ((), ())),
            preferred_element_type=jnp.float32,
        )[:, 0:1]

        @pl.when(kc == 0)
        def _():
            num_ref[b] = contrib_n
            l_ref[b] = contrib_l

        @pl.when(kc != 0)
        def _():
            num_ref[b] += contrib_n
            l_ref[b] += contrib_l

        def _rdma_pair(bb):
            rn = pltpu.make_async_remote_copy(
                src_ref=num_ref.at[bb], dst_ref=rnum_ref.at[bb],
                send_sem=nsend.at[bb], recv_sem=nrecv.at[bb],
                device_id=peer, device_id_type=pl.DeviceIdType.MESH,
            )
            rl_ = pltpu.make_async_remote_copy(
                src_ref=l_ref.at[bb], dst_ref=rl_ref.at[bb],
                send_sem=lsend.at[bb], recv_sem=lrecv.at[bb],
                device_id=peer, device_id_type=pl.DeviceIdType.MESH,
            )
            return rn, rl_

        @pl.when(kc == nkc - 1)
        def _():
            rn, rl_ = _rdma_pair(b)
            rn.start()
            rl_.start()

        @pl.when((b == B - 1) & (kc == nkc - 1))
        def _():
            for bb in range(B):
                rn, rl_ = _rdma_pair(bb)
                rn.wait_send()
                rl_.wait_send()
                rn.wait_recv()
                rl_.wait_recv()
            nsum = (num_ref[...] + rnum_ref[...]).reshape(B, QH, D)
            lsum = l_ref[...] + rl_ref[...]
            out_ref[...] = (nsum / lsum).reshape(B, SQ, H, D)

    return pl.pallas_call(
        body,
        grid=(B, nkc),
        in_specs=[
            pl.BlockSpec((None, SQ, H, D), lambda b, kc: (b, 0, 0, 0)),
            pl.BlockSpec((None, KC, H, D), lambda b, kc: (b, kc, 0, 0)),
            pl.BlockSpec((None, KC, H, D), lambda b, kc: (b, kc, 0, 0)),
        ],
        out_specs=pl.BlockSpec(memory_space=pltpu.VMEM),
        out_shape=jax.ShapeDtypeStruct((B, SQ, H, D), jnp.float32),
        scratch_shapes=[
            pltpu.VMEM((RH, QH), jnp.bfloat16),
            pltpu.VMEM((RH, QH), jnp.bfloat16),
            pltpu.VMEM((RH, 8), jnp.bfloat16),
            pltpu.VMEM((B, SQ, H, D), jnp.float32),
            pltpu.VMEM((B, QH, 1), jnp.float32),
            pltpu.VMEM((B, SQ, H, D), jnp.float32),
            pltpu.VMEM((B, QH, 1), jnp.float32),
            pltpu.SemaphoreType.DMA((B,)),
            pltpu.SemaphoreType.DMA((B,)),
            pltpu.SemaphoreType.DMA((B,)),
            pltpu.SemaphoreType.DMA((B,)),
        ],
        compiler_params=pltpu.CompilerParams(
            collective_id=0, vmem_limit_bytes=60 * 1024 * 1024
        ),
    )(Q, K, V)


def kernel(Q, K, V):
    return _fused(Q, K, V)


# device time: 55904 ns/iter; 3.7397x vs baseline; 1.0335x over previous
import jax
import jax.numpy as jnp
from jax import lax
from jax.experimental import pallas as pl
from jax.experimental.pallas import tpu as pltpu

B, SQ, H, D = 8, 8, 16, 128
QH = SQ * H
SCALE = D ** -0.5


def _flash_partial(Q, K, V):
    skv = K.shape[1]
    KC = 512
    nkc = skv // KC
    RH = KC * H

    def body(q_ref, k_ref, v_ref, num_ref, l_ref, pen_ref, p_ref, ones_ref):
        b = pl.program_id(0)
        kc = pl.program_id(1)

        @pl.when((b == 0) & (kc == 0))
        def _():
            r = lax.broadcasted_iota(jnp.int32, (RH, QH), 0)
            c = lax.broadcasted_iota(jnp.int32, (RH, QH), 1)
            pen_ref[...] = jnp.where((r % H) == (c % H), 0.0, -1e9).astype(
                jnp.float32
            )
            ones_ref[...] = jnp.ones((RH, 8), jnp.bfloat16)

        @pl.when(kc == 0)
        def _():
            num_ref[...] = jnp.zeros_like(num_ref)
            l_ref[...] = jnp.zeros_like(l_ref)

        q = q_ref[...].reshape(QH, D).astype(jnp.bfloat16)
        k = k_ref[...].reshape(RH, D).astype(jnp.bfloat16)
        g = lax.dot_general(
            k, q, (((1,), (1,)), ((), ())),
            preferred_element_type=jnp.float32,
        )
        p_ref[...] = jnp.exp(g * SCALE + pen_ref[...]).astype(jnp.bfloat16)

        p = p_ref[...]
        v = v_ref[...].reshape(RH, D).astype(jnp.bfloat16)
        num_ref[...] += lax.dot_general(
            p, v, (((0,), (0,)), ((), ())),
            preferred_element_type=jnp.float32,
        ).reshape(SQ, H, D)
        l_ref[...] += lax.dot_general(
            p, ones_ref[...], (((0,), (0,)), ((), ())),
            preferred_element_type=jnp.float32,
        )[:, 0:1]

    return pl.pallas_call(
        body,
        grid=(B, nkc),
        in_specs=[
            pl.BlockSpec((None, SQ, H, D), lambda b, kc: (b, 0, 0, 0)),
            pl.BlockSpec((None, KC, H, D), lambda b, kc: (b, kc, 0, 0)),
            pl.BlockSpec((None, KC, H, D), lambda b, kc: (b, kc, 0, 0)),
        ],
        out_specs=[
            pl.BlockSpec((None, SQ, H, D), lambda b, kc: (b, 0, 0, 0)),
            pl.BlockSpec((None, QH, 1), lambda b, kc: (b, 0, 0)),
        ],
        out_shape=[
            jax.ShapeDtypeStruct((B, SQ, H, D), jnp.float32),
            jax.ShapeDtypeStruct((B, QH, 1), jnp.float32),
        ],
        scratch_shapes=[
            pltpu.VMEM((RH, QH), jnp.float32),
            pltpu.VMEM((RH, QH), jnp.bfloat16),
            pltpu.VMEM((RH, 8), jnp.bfloat16),
        ],
    )(Q, K, V)


def _exchange_combine(num, l):

    def body(num_ref, l_ref, out_ref, rnum_ref, rl_ref,
             nsend, nrecv, lsend, lrecv):
        my_x = lax.axis_index("x")
        my_y = lax.axis_index("y")
        my_z = lax.axis_index("z")
        peer = (1 - my_x, my_y, my_z)

        barrier = pltpu.get_barrier_semaphore()
        pl.semaphore_signal(
            barrier, inc=1, device_id=peer,
            device_id_type=pl.DeviceIdType.MESH,
        )
        pl.semaphore_wait(barrier, 1)

        rdma_n = pltpu.make_async_remote_copy(
            src_ref=num_ref, dst_ref=rnum_ref,
            send_sem=nsend, recv_sem=nrecv,
            device_id=peer, device_id_type=pl.DeviceIdType.MESH,
        )
        rdma_l = pltpu.make_async_remote_copy(
            src_ref=l_ref, dst_ref=rl_ref,
            send_sem=lsend, recv_sem=lrecv,
            device_id=peer, device_id_type=pl.DeviceIdType.MESH,
        )
        rdma_n.start()
        rdma_l.start()
        rdma_n.wait()
        rdma_l.wait()

        nsum = (num_ref[...] + rnum_ref[...]).reshape(B, QH, D)
        lsum = l_ref[...] + rl_ref[...]
        out_ref[...] = (nsum / lsum).reshape(B, SQ, H, D)

    return pl.pallas_call(
        body,
        in_specs=[
            pl.BlockSpec(memory_space=pltpu.VMEM),
            pl.BlockSpec(memory_space=pltpu.VMEM),
        ],
        out_specs=pl.BlockSpec(memory_space=pltpu.VMEM),
        out_shape=jax.ShapeDtypeStruct((B, SQ, H, D), jnp.float32),
        scratch_shapes=[
            pltpu.VMEM((B, SQ, H, D), jnp.float32),
            pltpu.VMEM((B, QH, 1), jnp.float32),
            pltpu.SemaphoreType.DMA,
            pltpu.SemaphoreType.DMA,
            pltpu.SemaphoreType.DMA,
            pltpu.SemaphoreType.DMA,
        ],
        compiler_params=pltpu.CompilerParams(collective_id=0),
    )(num, l)


def _fused(Q, K, V):
    skv = K.shape[1]
    KC = 1024
    nkc = skv // KC
    RH = KC * H

    def body(q_ref, k_ref, v_ref, out_ref,
             pen_ref, ones_ref, num_ref, l_ref, rnum_ref, rl_ref,
             nsend, nrecv, lsend, lrecv):
        b = pl.program_id(0)
        kc = pl.program_id(1)
        my_x = lax.axis_index("x")
        my_y = lax.axis_index("y")
        my_z = lax.axis_index("z")
        peer = (1 - my_x, my_y, my_z)

        @pl.when((b == 0) & (kc == 0))
        def _():
            r = lax.broadcasted_iota(jnp.int32, (RH, QH), 0)
            c = lax.broadcasted_iota(jnp.int32, (RH, QH), 1)
            pen_ref[...] = jnp.where((r % H) == (c % H), 0.0, -1e9).astype(
                jnp.bfloat16
            )
            ones_ref[...] = jnp.ones((RH, 8), jnp.bfloat16)
            barrier = pltpu.get_barrier_semaphore()
            pl.semaphore_signal(
                barrier, inc=1, device_id=peer,
                device_id_type=pl.DeviceIdType.MESH,
            )
            pl.semaphore_wait(barrier, 1)

        q = (q_ref[...].reshape(QH, D) * SCALE).astype(jnp.bfloat16)
        k = k_ref[...].reshape(RH, D).astype(jnp.bfloat16)
        g = lax.dot_general(
            k, q, (((1,), (1,)), ((), ())),
            preferred_element_type=jnp.float32,
        )
        p = jnp.exp(g.astype(jnp.bfloat16) + pen_ref[...])
        v = v_ref[...].reshape(RH, D).astype(jnp.bfloat16)
        contrib_n = lax.dot_general(
            p, v, (((0,), (0,)), ((), ())),
            preferred_element_type=jnp.float32,
        ).reshape(SQ, H, D)
        contrib_l = lax.dot_general(
            p, ones_ref[...], (((0,), (0,)), ((), ())),
            preferred_element_type=jnp.float32,
        )[:, 0:1]

        @pl.when(kc == 0)
        def _():
            num_ref[b] = contrib_n
            l_ref[b] = contrib_l

        @pl.when(kc != 0)
        def _():
            num_ref[b] += contrib_n
            l_ref[b] += contrib_l

        def _rdma_pair(bb):
            rn = pltpu.make_async_remote_copy(
                src_ref=num_ref.at[bb], dst_ref=rnum_ref.at[bb],
                send_sem=nsend.at[bb], recv_sem=nrecv.at[bb],
                device_id=peer, device_id_type=pl.DeviceIdType.MESH,
            )
            rl_ = pltpu.make_async_remote_copy(
                src_ref=l_ref.at[bb], dst_ref=rl_ref.at[bb],
                send_sem=lsend.at[bb], recv_sem=lrecv.at[bb],
                device_id=peer, device_id_type=pl.DeviceIdType.MESH,
            )
            return rn, rl_

        @pl.when(kc == nkc - 1)
        def _():
            rn, rl_ = _rdma_pair(b)
            rn.start()
            rl_.start()

        @pl.when((b == B - 1) & (kc == nkc - 1))
        def _():
            for bb in range(B):
                rn, rl_ = _rdma_pair(bb)
                rn.wait_send()
                rl_.wait_send()
                rn.wait_recv()
                rl_.wait_recv()
            nsum = (num_ref[...] + rnum_ref[...]).reshape(B, QH, D)
            lsum = l_ref[...] + rl_ref[...]
            out_ref[...] = (nsum / lsum).reshape(B, SQ, H, D)

    return pl.pallas_call(
        body,
        grid=(B, nkc),
        in_specs=[
            pl.BlockSpec((None, SQ, H, D), lambda b, kc: (b, 0, 0, 0)),
            pl.BlockSpec((None, KC, H, D), lambda b, kc: (b, kc, 0, 0)),
            pl.BlockSpec((None, KC, H, D), lambda b, kc: (b, kc, 0, 0)),
        ],
        out_specs=pl.BlockSpec(memory_space=pltpu.VMEM),
        out_shape=jax.ShapeDtypeStruct((B, SQ, H, D), jnp.float32),
        scratch_shapes=[
            pltpu.VMEM((RH, QH), jnp.bfloat16),
            pltpu.VMEM((RH, 8), jnp.bfloat16),
            pltpu.VMEM((B, SQ, H, D), jnp.float32),
            pltpu.VMEM((B, QH, 1), jnp.float32),
            pltpu.VMEM((B, SQ, H, D), jnp.float32),
            pltpu.VMEM((B, QH, 1), jnp.float32),
            pltpu.SemaphoreType.DMA((B,)),
            pltpu.SemaphoreType.DMA((B,)),
            pltpu.SemaphoreType.DMA((B,)),
            pltpu.SemaphoreType.DMA((B,)),
        ],
        compiler_params=pltpu.CompilerParams(
            collective_id=0, vmem_limit_bytes=60 * 1024 * 1024
        ),
    )(Q, K, V)


def _stream_floor(Q, K, V):
    skv = K.shape[1]

    def body(q_ref, k_ref, v_ref, out_ref):
        b = pl.program_id(0)
        out_ref[b] = k_ref[:SQ] + v_ref[:SQ]

    return pl.pallas_call(
        body,
        grid=(B,),
        in_specs=[
            pl.BlockSpec((None, SQ, H, D), lambda b: (b, 0, 0, 0)),
            pl.BlockSpec((None, skv, H, D), lambda b: (b, 0, 0, 0)),
            pl.BlockSpec((None, skv, H, D), lambda b: (b, 0, 0, 0)),
        ],
        out_specs=pl.BlockSpec(memory_space=pltpu.VMEM),
        out_shape=jax.ShapeDtypeStruct((B, SQ, H, D), jnp.float32),
        compiler_params=pltpu.CompilerParams(
            vmem_limit_bytes=60 * 1024 * 1024
        ),
    )(Q, K, V)


def kernel(Q, K, V):
    return _fused(Q, K, V)


# device time: 55625 ns/iter; 3.7584x vs baseline; 1.0050x over previous
import jax
import jax.numpy as jnp
from jax import lax
from jax.experimental import pallas as pl
from jax.experimental.pallas import tpu as pltpu

B, SQ, H, D = 8, 8, 16, 128
QH = SQ * H
SCALE = D ** -0.5


def _flash_partial(Q, K, V):
    skv = K.shape[1]
    KC = 512
    nkc = skv // KC
    RH = KC * H

    def body(q_ref, k_ref, v_ref, num_ref, l_ref, pen_ref, p_ref, ones_ref):
        b = pl.program_id(0)
        kc = pl.program_id(1)

        @pl.when((b == 0) & (kc == 0))
        def _():
            r = lax.broadcasted_iota(jnp.int32, (RH, QH), 0)
            c = lax.broadcasted_iota(jnp.int32, (RH, QH), 1)
            pen_ref[...] = jnp.where((r % H) == (c % H), 0.0, -1e9).astype(
                jnp.float32
            )
            ones_ref[...] = jnp.ones((RH, 8), jnp.bfloat16)

        @pl.when(kc == 0)
        def _():
            num_ref[...] = jnp.zeros_like(num_ref)
            l_ref[...] = jnp.zeros_like(l_ref)

        q = q_ref[...].reshape(QH, D).astype(jnp.bfloat16)
        k = k_ref[...].reshape(RH, D).astype(jnp.bfloat16)
        g = lax.dot_general(
            k, q, (((1,), (1,)), ((), ())),
            preferred_element_type=jnp.float32,
        )
        p_ref[...] = jnp.exp(g * SCALE + pen_ref[...]).astype(jnp.bfloat16)

        p = p_ref[...]
        v = v_ref[...].reshape(RH, D).astype(jnp.bfloat16)
        num_ref[...] += lax.dot_general(
            p, v, (((0,), (0,)), ((), ())),
            preferred_element_type=jnp.float32,
        ).reshape(SQ, H, D)
        l_ref[...] += lax.dot_general(
            p, ones_ref[...], (((0,), (0,)), ((), ())),
            preferred_element_type=jnp.float32,
        )[:, 0:1]

    return pl.pallas_call(
        body,
        grid=(B, nkc),
        in_specs=[
            pl.BlockSpec((None, SQ, H, D), lambda b, kc: (b, 0, 0, 0)),
            pl.BlockSpec((None, KC, H, D), lambda b, kc: (b, kc, 0, 0)),
            pl.BlockSpec((None, KC, H, D), lambda b, kc: (b, kc, 0, 0)),
        ],
        out_specs=[
            pl.BlockSpec((None, SQ, H, D), lambda b, kc: (b, 0, 0, 0)),
            pl.BlockSpec((None, QH, 1), lambda b, kc: (b, 0, 0)),
        ],
        out_shape=[
            jax.ShapeDtypeStruct((B, SQ, H, D), jnp.float32),
            jax.ShapeDtypeStruct((B, QH, 1), jnp.float32),
        ],
        scratch_shapes=[
            pltpu.VMEM((RH, QH), jnp.float32),
            pltpu.VMEM((RH, QH), jnp.bfloat16),
            pltpu.VMEM((RH, 8), jnp.bfloat16),
        ],
    )(Q, K, V)


def _exchange_combine(num, l):

    def body(num_ref, l_ref, out_ref, rnum_ref, rl_ref,
             nsend, nrecv, lsend, lrecv):
        my_x = lax.axis_index("x")
        my_y = lax.axis_index("y")
        my_z = lax.axis_index("z")
        peer = (1 - my_x, my_y, my_z)

        barrier = pltpu.get_barrier_semaphore()
        pl.semaphore_signal(
            barrier, inc=1, device_id=peer,
            device_id_type=pl.DeviceIdType.MESH,
        )
        pl.semaphore_wait(barrier, 1)

        rdma_n = pltpu.make_async_remote_copy(
            src_ref=num_ref, dst_ref=rnum_ref,
            send_sem=nsend, recv_sem=nrecv,
            device_id=peer, device_id_type=pl.DeviceIdType.MESH,
        )
        rdma_l = pltpu.make_async_remote_copy(
            src_ref=l_ref, dst_ref=rl_ref,
            send_sem=lsend, recv_sem=lrecv,
            device_id=peer, device_id_type=pl.DeviceIdType.MESH,
        )
        rdma_n.start()
        rdma_l.start()
        rdma_n.wait()
        rdma_l.wait()

        nsum = (num_ref[...] + rnum_ref[...]).reshape(B, QH, D)
        lsum = l_ref[...] + rl_ref[...]
        out_ref[...] = (nsum / lsum).reshape(B, SQ, H, D)

    return pl.pallas_call(
        body,
        in_specs=[
            pl.BlockSpec(memory_space=pltpu.VMEM),
            pl.BlockSpec(memory_space=pltpu.VMEM),
        ],
        out_specs=pl.BlockSpec(memory_space=pltpu.VMEM),
        out_shape=jax.ShapeDtypeStruct((B, SQ, H, D), jnp.float32),
        scratch_shapes=[
            pltpu.VMEM((B, SQ, H, D), jnp.float32),
            pltpu.VMEM((B, QH, 1), jnp.float32),
            pltpu.SemaphoreType.DMA,
            pltpu.SemaphoreType.DMA,
            pltpu.SemaphoreType.DMA,
            pltpu.SemaphoreType.DMA,
        ],
        compiler_params=pltpu.CompilerParams(collective_id=0),
    )(num, l)


def _fused(Q, K, V):
    skv = K.shape[1]
    KC = skv
    RH = KC * H

    def body(q_ref, k_ref, v_ref, out_ref,
             pen_ref, ones_ref, num_ref, l_ref, rnum_ref, rl_ref,
             nsend, nrecv, lsend, lrecv):
        b = pl.program_id(0)
        my_x = lax.axis_index("x")
        my_y = lax.axis_index("y")
        my_z = lax.axis_index("z")
        peer = (1 - my_x, my_y, my_z)

        @pl.when(b == 0)
        def _():
            r = lax.broadcasted_iota(jnp.int32, (RH, QH), 0)
            c = lax.broadcasted_iota(jnp.int32, (RH, QH), 1)
            pen_ref[...] = jnp.where((r % H) == (c % H), 0.0, -1e9).astype(
                jnp.bfloat16
            )
            ones_ref[...] = jnp.ones((RH, 8), jnp.bfloat16)
            barrier = pltpu.get_barrier_semaphore()
            pl.semaphore_signal(
                barrier, inc=1, device_id=peer,
                device_id_type=pl.DeviceIdType.MESH,
            )
            pl.semaphore_wait(barrier, 1)

        q = (q_ref[...].reshape(QH, D) * SCALE).astype(jnp.bfloat16)
        k = k_ref[...].reshape(RH, D).astype(jnp.bfloat16)
        g = lax.dot_general(
            k, q, (((1,), (1,)), ((), ())),
            preferred_element_type=jnp.float32,
        )
        p = jnp.exp(g.astype(jnp.bfloat16) + pen_ref[...])
        v = v_ref[...].reshape(RH, D).astype(jnp.bfloat16)
        contrib_n = lax.dot_general(
            p, v, (((0,), (0,)), ((), ())),
            preferred_element_type=jnp.float32,
        ).reshape(SQ, H, D)
        contrib_l = lax.dot_general(
            p, ones_ref[...], (((0,), (0,)), ((), ())),
            preferred_element_type=jnp.float32,
        )[:, 0:1]

        num_ref[b] = contrib_n
        l_ref[b] = contrib_l

        def _rdma_pair(bb):
            rn = pltpu.make_async_remote_copy(
                src_ref=num_ref.at[bb], dst_ref=rnum_ref.at[bb],
                send_sem=nsend.at[bb], recv_sem=nrecv.at[bb],
                device_id=peer, device_id_type=pl.DeviceIdType.MESH,
            )
            rl_ = pltpu.make_async_remote_copy(
                src_ref=l_ref.at[bb], dst_ref=rl_ref.at[bb],
                send_sem=lsend.at[bb], recv_sem=lrecv.at[bb],
                device_id=peer, device_id_type=pl.DeviceIdType.MESH,
            )
            return rn, rl_

        rn, rl_ = _rdma_pair(b)
        rn.start()
        rl_.start()

        @pl.when(b == B - 1)
        def _():
            for bb in range(B):
                rn, rl_ = _rdma_pair(bb)
                rn.wait_send()
                rl_.wait_send()
                rn.wait_recv()
                rl_.wait_recv()
            nsum = (num_ref[...] + rnum_ref[...]).reshape(B, QH, D)
            lsum = l_ref[...] + rl_ref[...]
            out_ref[...] = (nsum / lsum).reshape(B, SQ, H, D)

    return pl.pallas_call(
        body,
        grid=(B,),
        in_specs=[
            pl.BlockSpec((None, SQ, H, D), lambda b: (b, 0, 0, 0)),
            pl.BlockSpec((None, KC, H, D), lambda b: (b, 0, 0, 0)),
            pl.BlockSpec((None, KC, H, D), lambda b: (b, 0, 0, 0)),
        ],
        out_specs=pl.BlockSpec(memory_space=pltpu.VMEM),
        out_shape=jax.ShapeDtypeStruct((B, SQ, H, D), jnp.float32),
        scratch_shapes=[
            pltpu.VMEM((RH, QH), jnp.bfloat16),
            pltpu.VMEM((RH, 8), jnp.bfloat16),
            pltpu.VMEM((B, SQ, H, D), jnp.float32),
            pltpu.VMEM((B, QH, 1), jnp.float32),
            pltpu.VMEM((B, SQ, H, D), jnp.float32),
            pltpu.VMEM((B, QH, 1), jnp.float32),
            pltpu.SemaphoreType.DMA((B,)),
            pltpu.SemaphoreType.DMA((B,)),
            pltpu.SemaphoreType.DMA((B,)),
            pltpu.SemaphoreType.DMA((B,)),
        ],
        compiler_params=pltpu.CompilerParams(
            collective_id=0, vmem_limit_bytes=60 * 1024 * 1024
        ),
    )(Q, K, V)


def _stream_floor(Q, K, V):
    skv = K.shape[1]

    def body(q_ref, k_ref, v_ref, out_ref):
        b = pl.program_id(0)
        out_ref[b] = k_ref[:SQ] + v_ref[:SQ]

    return pl.pallas_call(
        body,
        grid=(B,),
        in_specs=[
            pl.BlockSpec((None, SQ, H, D), lambda b: (b, 0, 0, 0)),
            pl.BlockSpec((None, skv, H, D), lambda b: (b, 0, 0, 0)),
            pl.BlockSpec((None, skv, H, D), lambda b: (b, 0, 0, 0)),
        ],
        out_specs=pl.BlockSpec(memory_space=pltpu.VMEM),
        out_shape=jax.ShapeDtypeStruct((B, SQ, H, D), jnp.float32),
        compiler_params=pltpu.CompilerParams(
            vmem_limit_bytes=60 * 1024 * 1024
        ),
    )(Q, K, V)


def kernel(Q, K, V):
    return _fused(Q, K, V)
